# Initial kernel scaffold; baseline (speedup 1.0000x reference)
import jax
import jax.numpy as jnp
from jax.experimental import pallas as pl


def kernel(x):
    raise NotImplementedError("write your pallas kernel here")


# speedup vs baseline: 23.2714x; 23.2714x over previous
import functools

import jax
import jax.numpy as jnp
from jax import lax
from jax.experimental import pallas as pl
from jax.experimental.pallas import tpu as pltpu
from jax.experimental.pallas import tpu_sc as plsc

_H = 224
_W = 224
_STRIDE = 8
_PATCH = 16
_B = 4
_C = 32
_NY = 27
_NX = 27
_PXC = _PATCH * _C
_UNITS = _W * _C // 256
_ROWS_PER_WORKER = (_B * _H) // 32

_mesh = plsc.VectorSubcoreMesh(core_axis_name="c", subcore_axis_name="s")


@functools.partial(
    pl.kernel,
    out_type=jax.ShapeDtypeStruct((_B, _H, _UNITS, 256), jnp.float32),
    mesh=_mesh,
    scratch_types=[
        pltpu.VMEM((_NX + 2, _PXC), jnp.float32),
        pltpu.VMEM((_NX + 2, _PXC), jnp.float32),
        pltpu.VMEM((_UNITS, 256), jnp.float32),
    ],
    compiler_params=pltpu.CompilerParams(use_tc_tiling_on_sc=False),
)
def _overlap_add_sc(x_ref, out_ref, pa, pb, ob):
    cid = lax.axis_index("c")
    sid = lax.axis_index("s")
    wid = cid * 16 + sid
    bidx = wid // 8
    h0 = (wid % 8) * _ROWS_PER_WORKER

    def row_body(i, carry):
        h = h0 + i
        iy_a = jnp.minimum(lax.div(h, 8), _NY - 1)
        py_a = h - 8 * iy_a
        iy_b = jnp.maximum(iy_a - 1, 0)
        py_b = jnp.minimum(py_a + 8, _PATCH - 1)

        pltpu.sync_copy(x_ref.at[bidx, iy_a, :, py_a, :], pa.at[pl.ds(1, _NX)])
        pltpu.sync_copy(x_ref.at[bidx, iy_b, :, py_b, :], pb.at[pl.ds(1, _NX)])

        for v in range(16):
            pa[0, pl.ds(256 + v * 16, 16)] = pa[1, pl.ds(v * 16, 16)]
            pa[_NX + 1, pl.ds(v * 16, 16)] = pa[_NX, pl.ds(256 + v * 16, 16)]
            pb[0, pl.ds(256 + v * 16, 16)] = pb[1, pl.ds(v * 16, 16)]
            pb[_NX + 1, pl.ds(v * 16, 16)] = pb[_NX, pl.ds(256 + v * 16, 16)]

        two_y = jnp.logical_and(h >= _STRIDE, h < _H - _STRIDE)
        wy = jnp.where(two_y, 1.0, 0.0).astype(jnp.float32)
        sc = jnp.where(two_y, 0.25, 0.5).astype(jnp.float32)
        wyv = jnp.full((16,), wy, jnp.float32)
        scv = jnp.full((16,), sc, jnp.float32)

        def unit_body(r, c2):
            for v in range(16):
                a = pa[r + 1, pl.ds(v * 16, 16)] + pa[r, pl.ds(256 + v * 16, 16)]
                b = pb[r + 1, pl.ds(v * 16, 16)] + pb[r, pl.ds(256 + v * 16, 16)]
                ob[r, pl.ds(v * 16, 16)] = (a + wyv * b) * scv
            return c2

        lax.fori_loop(0, _UNITS, unit_body, 0)
        pltpu.sync_copy(ob, out_ref.at[bidx, h])
        return carry

    lax.fori_loop(0, _ROWS_PER_WORKER, row_body, 0)


def kernel(x):
    xr = x.reshape(_B, _NY, _NX, _PATCH, _PXC)
    out = _overlap_add_sc(xr)
    return out.reshape(_B, _H, _W, _C)


# speedup vs baseline: 25.3118x; 1.0877x over previous
import functools

import jax
import jax.numpy as jnp
from jax import lax
from jax.experimental import pallas as pl
from jax.experimental.pallas import tpu as pltpu
from jax.experimental.pallas import tpu_sc as plsc

_H = 224
_W = 224
_STRIDE = 8
_PATCH = 16
_B = 4
_C = 32
_NY = 27
_NX = 27
_PXC = _PATCH * _C
_UNITS = _W * _C // 256
_ROWS_PER_WORKER = (_B * _H) // 32

_mesh = plsc.VectorSubcoreMesh(core_axis_name="c", subcore_axis_name="s")


@functools.partial(
    pl.kernel,
    out_type=jax.ShapeDtypeStruct((_B, _H, _UNITS, 256), jnp.float32),
    mesh=_mesh,
    scratch_types=[
        pltpu.VMEM((2, _NX + 2, _PXC), jnp.float32),
        pltpu.VMEM((2, _NX + 2, _PXC), jnp.float32),
        pltpu.VMEM((2, _UNITS, 256), jnp.float32),
        pltpu.SemaphoreType.DMA((2,)),
        pltpu.SemaphoreType.DMA((2,)),
        pltpu.SemaphoreType.DMA((2,)),
    ],
    compiler_params=pltpu.CompilerParams(use_tc_tiling_on_sc=False),
)
def _overlap_add_sc(x_ref, out_ref, pa, pb, ob, sema, semb, semo):
    cid = lax.axis_index("c")
    sid = lax.axis_index("s")
    wid = cid * 16 + sid
    bidx = wid // 8
    h0 = (wid % 8) * _ROWS_PER_WORKER

    def in_copies(row, j):
        h = h0 + row
        iy_a = jnp.minimum(lax.div(h, 8), _NY - 1)
        py_a = h - 8 * iy_a
        iy_b = jnp.maximum(iy_a - 1, 0)
        py_b = jnp.minimum(py_a + 8, _PATCH - 1)
        ca = pltpu.make_async_copy(
            x_ref.at[bidx, iy_a, :, py_a, :], pa.at[j, pl.ds(1, _NX)], sema.at[j])
        cb = pltpu.make_async_copy(
            x_ref.at[bidx, iy_b, :, py_b, :], pb.at[j, pl.ds(1, _NX)], semb.at[j])
        return ca, cb

    def out_copy(row, j):
        h = h0 + row
        return pltpu.make_async_copy(ob.at[j], out_ref.at[bidx, h], semo.at[j])

    ca0, cb0 = in_copies(0, 0)
    ca0.start()
    cb0.start()

    def row_body(i, carry):
        j = lax.rem(i, 2)
        h = h0 + i

        @pl.when(i + 1 < _ROWS_PER_WORKER)
        def _prefetch():
            ca, cb = in_copies(i + 1, 1 - j)
            ca.start()
            cb.start()

        ca, cb = in_copies(i, j)
        ca.wait()
        cb.wait()

        for v in range(16):
            pa[j, 0, pl.ds(256 + v * 16, 16)] = pa[j, 1, pl.ds(v * 16, 16)]
            pa[j, _NX + 1, pl.ds(v * 16, 16)] = pa[j, _NX, pl.ds(256 + v * 16, 16)]
            pb[j, 0, pl.ds(256 + v * 16, 16)] = pb[j, 1, pl.ds(v * 16, 16)]
            pb[j, _NX + 1, pl.ds(v * 16, 16)] = pb[j, _NX, pl.ds(256 + v * 16, 16)]

        two_y = jnp.logical_and(h >= _STRIDE, h < _H - _STRIDE)
        wy = jnp.where(two_y, 1.0, 0.0).astype(jnp.float32)
        sc = jnp.where(two_y, 0.25, 0.5).astype(jnp.float32)
        wyv = jnp.full((16,), wy, jnp.float32)
        scv = jnp.full((16,), sc, jnp.float32)

        @pl.when(i >= 2)
        def _drain_out():
            out_copy(i, j).wait()

        def unit_body(r, c2):
            for v in range(16):
                a = pa[j, r + 1, pl.ds(v * 16, 16)] + pa[j, r, pl.ds(256 + v * 16, 16)]
                b = pb[j, r + 1, pl.ds(v * 16, 16)] + pb[j, r, pl.ds(256 + v * 16, 16)]
                ob[j, r, pl.ds(v * 16, 16)] = (a + wyv * b) * scv
            return c2

        lax.fori_loop(0, _UNITS, unit_body, 0)
        out_copy(i, j).start()
        return carry

    lax.fori_loop(0, _ROWS_PER_WORKER, row_body, 0)

    out_copy(_ROWS_PER_WORKER - 2, lax.rem(_ROWS_PER_WORKER - 2, 2)).wait()
    out_copy(_ROWS_PER_WORKER - 1, lax.rem(_ROWS_PER_WORKER - 1, 2)).wait()


def kernel(x):
    xr = x.reshape(_B, _NY, _NX, _PATCH, _PXC)
    out = _overlap_add_sc(xr)
    return out.reshape(_B, _H, _W, _C)
